# baseline (speedup 1.0000x reference)
import jax
import jax.numpy as jnp
from jax.experimental import pallas as pl
from jax.experimental.pallas import tpu as pltpu

N = 10000
E = 320000
H = 8
D = 128
DV = D // H
RPD = 32
VOCAB = 21
EPS = 1e-12


def _leaky(a):
    return jnp.where(a > 0, a, 0.2 * a)


def _final_proj_body(cat_ref, fpw_ref, fpb_ref, out_ref):
    cat = cat_ref[...]
    out = jnp.dot(cat, fpw_ref[...], preferred_element_type=jnp.float32)
    out_ref[...] = out + fpb_ref[...]


def kernel(x, edge_index, rel_pos_idx, W_value, rel_emb, W_relation,
           w_src, w_tgt, w_rel, fp_w, fp_b):
    src = edge_index[0]
    tgt = edge_index[1]
    v = (x @ W_value).reshape(N, H, DV)
    r = (jnp.take(rel_emb, rel_pos_idx, axis=0) @ W_relation).reshape(E, H, RPD // H)
    s_src = (v * w_src).sum(-1)
    s_tgt = (v * w_tgt).sum(-1)
    scores_v = _leaky(jnp.take(s_src, src, axis=0) + jnp.take(s_tgt, tgt, axis=0))
    scores_r = _leaky((r * w_rel).sum(-1))
    v_src = jnp.take(v, src, axis=0)

    def agg(scores):
        scaled = scores - scores.max()
        exp = jnp.exp(scaled)
        denom = jax.ops.segment_sum(exp, tgt, num_segments=N)
        attn = exp / (jnp.take(denom, tgt, axis=0) + EPS)
        return jax.ops.segment_sum(attn[:, :, None] * v_src, tgt, num_segments=N)

    out_value = agg(scores_v)
    out_rel = agg(scores_r)
    cat = jnp.concatenate([out_value, out_rel], axis=-1).reshape(N, H * 2 * DV)

    eye = jnp.eye(H, dtype=jnp.float32)
    fpw_bd = (eye[:, None, :, None] * fp_w.transpose(0, 1, 2)[:, :, None, :]
              ).reshape(H * 2 * DV, H * DV)
    fpb_flat = fp_b.reshape(1, H * DV)

    BLK = 2000
    return pl.pallas_call(
        _final_proj_body,
        grid=(N // BLK,),
        in_specs=[
            pl.BlockSpec((BLK, H * 2 * DV), lambda i: (i, 0)),
            pl.BlockSpec((H * 2 * DV, H * DV), lambda i: (0, 0)),
            pl.BlockSpec((1, H * DV), lambda i: (0, 0)),
        ],
        out_specs=pl.BlockSpec((BLK, H * DV), lambda i: (i, 0)),
        out_shape=jax.ShapeDtypeStruct((N, H * DV), jnp.float32),
    )(cat, fpw_bd, fpb_flat)


# speedup vs baseline: 52.3471x; 52.3471x over previous
import functools

import jax
import jax.numpy as jnp
from jax import lax
from jax.experimental import pallas as pl
from jax.experimental.pallas import tpu as pltpu
from jax.experimental.pallas import tpu_sc as plsc

N = 10000
E = 320000
H = 8
D = 128
DV = D // H
RPD = 32
RDV = RPD // H
VOCAB = 21
EPS = 1e-12

NC = 2
NS = 16
L = 16
HG = H // NC

ROWW = 80
ACCW = 144
EPT = E // NS
C = 80
NCHUNK = EPT // C
NPT = N // NS


def _leaky(a):
    return jnp.where(a > 0, a, 0.2 * a)


PBLK = 2000


def _prep_body(x_ref, wv_ref, ws_ref, wt_ref, vall_ref, st_ref):
    v2 = jnp.dot(x_ref[...], wv_ref[...], preferred_element_type=jnp.float32)
    v3 = v2.reshape(PBLK, H, DV)
    s_src = (v3 * ws_ref[...]).sum(-1)
    s_tgt = (v3 * wt_ref[...]).sum(-1)
    z12 = jnp.zeros((PBLK, 12), jnp.float32)
    vall_ref[...] = jnp.stack([
        jnp.concatenate([v2[:, :64], s_src[:, 0:4], z12], axis=1),
        jnp.concatenate([v2[:, 64:], s_src[:, 4:8], z12], axis=1)], axis=0)
    st_ref[...] = jnp.concatenate([s_tgt, s_src], axis=1)


def _tbl_body(st_ref, re_ref, wr_ref, wrel_ref, et_ref, mv_ref):
    colmax = jnp.max(st_ref[...], axis=0)
    mv = _leaky(jnp.max(colmax[8:16] + colmax[0:8]))
    r_tbl = jnp.dot(re_ref[...], wr_ref[...], preferred_element_type=jnp.float32)
    sr = _leaky((r_tbl.reshape(VOCAB, H, RDV) * wrel_ref[...]).sum(-1))
    etbl = jnp.exp(sr - jnp.max(sr))
    et_ref[...] = jnp.concatenate([etbl, jnp.zeros((32 - VOCAB, H), jnp.float32)], axis=0)
    mv_ref[...] = jnp.full((1, 128), mv, jnp.float32)


def _edge_body(vall_hbm, sttgt_hbm, et_hbm, mv_hbm, src_hbm, tgt_hbm, rel_hbm,
               out_hbm,
               srcp, tgtb, relb, rows, trow, contrib, et_v, mv_v, acc,
               sem1, sem2):
    c = lax.axis_index("c")
    s = lax.axis_index("s")

    pltpu.sync_copy(et_hbm, et_v)
    pltpu.sync_copy(mv_hbm, mv_v)

    zero = jnp.zeros((L,), jnp.float32)
    lanes = lax.iota(jnp.int32, L)
    cN = c * N
    mvec = mv_v[0, pl.ds(0, L)]

    def _zc(i, carry):
        for k in range(ACCW // L):
            contrib[i, pl.ds(k * L, L)] = zero
        return carry
    lax.fori_loop(0, C, _zc, 0)

    nb = s * NPT
    for k in range(NPT // C):
        pltpu.sync_copy(contrib, acc.at[pl.ds(nb + k * C, C)])
    rem = NPT % C
    if rem:
        pltpu.sync_copy(contrib.at[pl.ds(0, rem)],
                        acc.at[pl.ds(nb + (NPT // C) * C, rem)])
    plsc.subcore_barrier()

    def _chunk(g, carry):
        pltpu.sync_copy(src_hbm.at[s, g], srcp)
        pltpu.sync_copy(tgt_hbm.at[s, g], tgtb)
        pltpu.sync_copy(rel_hbm.at[s, g], relb)
        for k in range(C // L):
            sl = pl.ds(k * L, L)
            srcp[sl] = srcp[sl] + cN
        gd = pltpu.async_copy(vall_hbm.at[srcp], rows, sem1)
        td = pltpu.async_copy(sttgt_hbm.at[tgtb], trow, sem2)
        gd.wait()
        td.wait()
        for gg in range(C // L):
            elanes = lanes + (gg * L)
            rel16 = relb[pl.ds(gg * L, L)]
            for h in range(HG):
                hcol = jnp.zeros((L,), jnp.int32) + (c * HG + h)
                a = plsc.load_gather(rows, [elanes, jnp.full((L,), 64 + h, jnp.int32)])
                b = plsc.load_gather(trow, [elanes, hcol])
                sv = a + b
                sv = jnp.where(sv > 0, sv, 0.2 * sv)
                w1 = jnp.exp(sv - mvec)
                w2 = plsc.load_gather(et_v, [rel16, hcol])
                plsc.store_scatter(contrib, [elanes, jnp.full((L,), 128 + h, jnp.int32)], w1)
                plsc.store_scatter(contrib, [elanes, jnp.full((L,), 132 + h, jnp.int32)], w2)
        def _pedge(e, carry2):
            erow = jnp.zeros((L,), jnp.int32) + e
            for h in range(HG):
                w1b = plsc.load_gather(contrib, [erow, jnp.full((L,), 128 + h, jnp.int32)])
                w2b = plsc.load_gather(contrib, [erow, jnp.full((L,), 132 + h, jnp.int32)])
                vs = rows[e, pl.ds(h * L, L)]
                contrib[e, pl.ds(h * L, L)] = w1b * vs
                contrib[e, pl.ds(64 + h * L, L)] = w2b * vs
            return carry2
        lax.fori_loop(0, C, _pedge, 0)
        pltpu.sync_copy(contrib, acc.at[tgtb], add=True)
        return carry
    lax.fori_loop(0, NCHUNK, _chunk, 0)

    plsc.subcore_barrier()
    for k in range(NPT // C):
        pltpu.sync_copy(acc.at[pl.ds(nb + k * C, C)],
                        out_hbm.at[c, pl.ds(nb + k * C, C)])
    if rem:
        pltpu.sync_copy(acc.at[pl.ds(nb + (NPT // C) * C, rem)],
                        out_hbm.at[c, pl.ds(nb + (NPT // C) * C, rem)])


_edge_kernel = functools.partial(
    pl.kernel,
    out_type=jax.ShapeDtypeStruct((NC, N, ACCW), jnp.float32),
    mesh=plsc.VectorSubcoreMesh(core_axis_name="c", subcore_axis_name="s"),
    scratch_types=[
        pltpu.VMEM((C,), jnp.int32),
        pltpu.VMEM((C,), jnp.int32),
        pltpu.VMEM((C,), jnp.int32),
        pltpu.VMEM((C, ROWW), jnp.float32),
        pltpu.VMEM((C, 16), jnp.float32),
        pltpu.VMEM((C, ACCW), jnp.float32),
        pltpu.VMEM((32, H), jnp.float32),
        pltpu.VMEM((1, 128), jnp.float32),
        pltpu.VMEM_SHARED((N, ACCW), jnp.float32),
        pltpu.SemaphoreType.DMA,
        pltpu.SemaphoreType.DMA,
    ],
    compiler_params=pltpu.CompilerParams(use_tc_tiling_on_sc=False,
                                         needs_layout_passes=False),
)(_edge_body)


def _final_body(acc_ref, fpw_ref, fpb_ref, out_ref):
    a = acc_ref[...]
    parts = []
    for hh in range(H):
        cc, j = hh // HG, hh % HG
        den_v = a[cc, :, 128 + j:129 + j] + EPS
        den_r = a[cc, :, 132 + j:133 + j] + EPS
        parts.append(a[cc, :, j * 16:(j + 1) * 16] / den_v)
        parts.append(a[cc, :, 64 + j * 16:64 + (j + 1) * 16] / den_r)
    cat = jnp.concatenate(parts, axis=1)
    out_ref[...] = (jnp.dot(cat, fpw_ref[...], preferred_element_type=jnp.float32)
                    + fpb_ref[...])


def kernel(x, edge_index, rel_pos_idx, W_value, rel_emb, W_relation,
           w_src, w_tgt, w_rel, fp_w, fp_b):
    src_r = edge_index[0].reshape(NS, NCHUNK, C)
    tgt_r = edge_index[1].reshape(NS, NCHUNK, C)
    rel_r = rel_pos_idx.reshape(NS, NCHUNK, C)

    v_all3, st_tgt = pl.pallas_call(
        _prep_body,
        grid=(N // PBLK,),
        in_specs=[
            pl.BlockSpec((PBLK, D), lambda i: (i, 0)),
            pl.BlockSpec((D, D), lambda i: (0, 0)),
            pl.BlockSpec((1, H, DV), lambda i: (0, 0, 0)),
            pl.BlockSpec((1, H, DV), lambda i: (0, 0, 0)),
        ],
        out_specs=[
            pl.BlockSpec((2, PBLK, ROWW), lambda i: (0, i, 0)),
            pl.BlockSpec((PBLK, 16), lambda i: (i, 0)),
        ],
        out_shape=[
            jax.ShapeDtypeStruct((2, N, ROWW), jnp.float32),
            jax.ShapeDtypeStruct((N, 16), jnp.float32),
        ],
    )(x, W_value, w_src, w_tgt)
    v_all = v_all3.reshape(2 * N, ROWW)

    et, mv = pl.pallas_call(
        _tbl_body,
        out_shape=[
            jax.ShapeDtypeStruct((32, H), jnp.float32),
            jax.ShapeDtypeStruct((1, 128), jnp.float32),
        ],
    )(st_tgt, rel_emb, W_relation, w_rel)

    acc2 = _edge_kernel(v_all, st_tgt, et, mv, src_r, tgt_r, rel_r)

    eye = jnp.eye(H, dtype=jnp.float32)
    fpw_bd = (eye[:, None, :, None] * fp_w[:, :, None, :]).reshape(H * 2 * DV, H * DV)
    fpb_flat = fp_b.reshape(1, H * DV)

    BLK = 2000
    return pl.pallas_call(
        _final_body,
        grid=(N // BLK,),
        in_specs=[
            pl.BlockSpec((NC, BLK, ACCW), lambda i: (0, i, 0)),
            pl.BlockSpec((H * 2 * DV, H * DV), lambda i: (0, 0)),
            pl.BlockSpec((1, H * DV), lambda i: (0, 0)),
        ],
        out_specs=pl.BlockSpec((BLK, H * DV), lambda i: (i, 0)),
        out_shape=jax.ShapeDtypeStruct((N, H * DV), jnp.float32),
    )(acc2, fpw_bd, fpb_flat)
